# Initial kernel scaffold; baseline (speedup 1.0000x reference)
import functools

import jax
import jax.numpy as jnp
from jax import lax
from jax.experimental import pallas as pl
from jax.experimental.pallas import tpu as pltpu
from jax.experimental.pallas import tpu_sc as plsc

_NC = 2
_NS = 16
_NW = _NC * _NS
_CH = 80

_mesh = plsc.VectorSubcoreMesh(core_axis_name="c", subcore_axis_name="s")


def _deg_kernel(n, e):
    cpt = e // _CH // _NW
    zr = n // 5

    def body(dst_hbm, ones_hbm, z1_hbm, deg0, deg1, idx_all, ones_v, acc, sem):
        cid = lax.axis_index("c")
        sid = lax.axis_index("s")
        wid = cid * _NS + sid

        @pl.when(sid < 5)
        def _():
            pltpu.sync_copy(z1_hbm, acc.at[pl.ds(sid * zr, zr)])

        pltpu.sync_copy(ones_hbm, ones_v)
        pltpu.sync_copy(dst_hbm.at[pl.ds(wid * cpt, cpt)], idx_all)
        plsc.subcore_barrier()

        def chunk(j, carry):
            pltpu.sync_copy(ones_v, acc.at[idx_all.at[j]], add=True)
            return carry

        lax.fori_loop(0, cpt, chunk, 0)
        plsc.subcore_barrier()

        @pl.when(jnp.logical_and(sid < 5, cid == 0))
        def _():
            pltpu.sync_copy(acc.at[pl.ds(sid * zr, zr)],
                            deg0.at[pl.ds(sid * zr, zr)])

        @pl.when(jnp.logical_and(sid < 5, cid == 1))
        def _():
            pltpu.sync_copy(acc.at[pl.ds(sid * zr, zr)],
                            deg1.at[pl.ds(sid * zr, zr)])

    return pl.kernel(
        body,
        out_type=[jax.ShapeDtypeStruct((n,), jnp.float32),
                  jax.ShapeDtypeStruct((n,), jnp.float32)],
        mesh=_mesh,
        scratch_types=[
            pltpu.VMEM((cpt, _CH), jnp.int32),
            pltpu.VMEM((_CH,), jnp.float32),
            pltpu.VMEM_SHARED((n,), jnp.float32),
            pltpu.SemaphoreType.DMA,
        ],
    )


def _agg_kernel(n, e, d):
    cpt = e // _CH // _NW
    rpt = n // _NS
    zb = rpt // 5

    def body(src_hbm, dst_hbm, y_hbm, z2_hbm, out0, out1,
             idx_s, idx_d, buf, zbuf, acc, sem):
        cid = lax.axis_index("c")
        sid = lax.axis_index("s")
        wid = cid * _NS + sid

        pltpu.sync_copy(z2_hbm, zbuf)
        for k in range(5):
            pltpu.sync_copy(zbuf, acc.at[pl.ds(sid * rpt + k * zb, zb)])

        pltpu.sync_copy(src_hbm.at[pl.ds(wid * cpt, cpt)], idx_s)
        pltpu.sync_copy(dst_hbm.at[pl.ds(wid * cpt, cpt)], idx_d)
        plsc.subcore_barrier()

        def chunk(j, carry):
            pltpu.async_copy(y_hbm.at[idx_s.at[j]], buf, sem).wait()
            pltpu.sync_copy(buf, acc.at[idx_d.at[j]], add=True)
            return carry

        lax.fori_loop(0, cpt, chunk, 0)
        plsc.subcore_barrier()

        @pl.when(cid == 0)
        def _():
            pltpu.sync_copy(acc.at[pl.ds(sid * rpt, rpt)],
                            out0.at[pl.ds(sid * rpt, rpt)])

        @pl.when(cid == 1)
        def _():
            pltpu.sync_copy(acc.at[pl.ds(sid * rpt, rpt)],
                            out1.at[pl.ds(sid * rpt, rpt)])

    return pl.kernel(
        body,
        out_type=[jax.ShapeDtypeStruct((n, d), jnp.float32),
                  jax.ShapeDtypeStruct((n, d), jnp.float32)],
        mesh=_mesh,
        scratch_types=[
            pltpu.VMEM((cpt, _CH), jnp.int32),
            pltpu.VMEM((cpt, _CH), jnp.int32),
            pltpu.VMEM((_CH, d), jnp.float32),
            pltpu.VMEM((n // _NS // 5, d), jnp.float32),
            pltpu.VMEM_SHARED((n, d), jnp.float32),
            pltpu.SemaphoreType.DMA,
        ],
    )


def _y_body(x_ref, w_ref, d0_ref, d1_ref, y_ref, dinv_ref):
    deg = d0_ref[...] + d1_ref[...] + 1.0
    dv = lax.rsqrt(deg)
    xw = jnp.dot(x_ref[...], w_ref[...], preferred_element_type=jnp.float32)
    y_ref[...] = xw * dv
    dinv_ref[...] = dv


def _head_body(a0_ref, a1_ref, y_ref, dv_ref, bc_ref, wr_ref, br_ref, o_ref):
    s = a0_ref[...] + a1_ref[...] + y_ref[...]
    h = jnp.maximum(s * dv_ref[...] + bc_ref[...], 0.0)
    o_ref[...] = jnp.sum(h * wr_ref[...], axis=1, keepdims=True) + br_ref[...]


def kernel(x, edge_index, W_conv, b_conv, W_reg, b_reg):
    n, d = x.shape
    e = edge_index.shape[1]
    rows = e // _CH
    blk = n // 5

    src2 = edge_index[0].reshape(rows, _CH)
    dst2 = edge_index[1].reshape(rows, _CH)
    ones_ch = jnp.ones((_CH,), jnp.float32)
    z1 = jnp.zeros((n // 5,), jnp.float32)
    z2 = jnp.zeros((n // _NS // 5, d), jnp.float32)

    deg0, deg1 = _deg_kernel(n, e)(dst2, ones_ch, z1)

    y, dinv = pl.pallas_call(
        _y_body,
        grid=(n // blk,),
        in_specs=[
            pl.BlockSpec((blk, d), lambda i: (i, 0)),
            pl.BlockSpec((d, d), lambda i: (0, 0)),
            pl.BlockSpec((blk, 1), lambda i: (i, 0)),
            pl.BlockSpec((blk, 1), lambda i: (i, 0)),
        ],
        out_specs=[
            pl.BlockSpec((blk, d), lambda i: (i, 0)),
            pl.BlockSpec((blk, 1), lambda i: (i, 0)),
        ],
        out_shape=[
            jax.ShapeDtypeStruct((n, d), jnp.float32),
            jax.ShapeDtypeStruct((n, 1), jnp.float32),
        ],
    )(x, W_conv, deg0.reshape(n, 1), deg1.reshape(n, 1))

    acc0, acc1 = _agg_kernel(n, e, d)(src2, dst2, y, z2)

    out = pl.pallas_call(
        _head_body,
        grid=(n // blk,),
        in_specs=[
            pl.BlockSpec((blk, d), lambda i: (i, 0)),
            pl.BlockSpec((blk, d), lambda i: (i, 0)),
            pl.BlockSpec((blk, d), lambda i: (i, 0)),
            pl.BlockSpec((blk, 1), lambda i: (i, 0)),
            pl.BlockSpec((1, d), lambda i: (0, 0)),
            pl.BlockSpec((1, d), lambda i: (0, 0)),
            pl.BlockSpec((1, 1), lambda i: (0, 0)),
        ],
        out_specs=pl.BlockSpec((blk, 1), lambda i: (i, 0)),
        out_shape=jax.ShapeDtypeStruct((n, 1), jnp.float32),
    )(acc0, acc1, y, dinv, b_conv.reshape(1, d), W_reg.reshape(1, d),
      b_reg.reshape(1, 1))

    return out


# speedup vs baseline: 28.9367x; 28.9367x over previous
import functools

import jax
import jax.numpy as jnp
from jax import lax
from jax.experimental import pallas as pl
from jax.experimental.pallas import tpu as pltpu
from jax.experimental.pallas import tpu_sc as plsc

_NC = 2
_NS = 16
_NW = _NC * _NS
_CH = 125

_mesh = plsc.VectorSubcoreMesh(core_axis_name="c", subcore_axis_name="s")


def _deg_kernel(n, e):
    cpt = e // _CH // _NW

    def body(dst_hbm, ones_hbm, z1_hbm, deg0, deg1, idx_all, ones_v, stg, acc,
             sem):
        cid = lax.axis_index("c")
        sid = lax.axis_index("s")
        wid = cid * _NS + sid

        @pl.when(sid == 0)
        def _():
            pltpu.sync_copy(z1_hbm, stg)
            pltpu.sync_copy(stg, acc)

        pltpu.sync_copy(ones_hbm, ones_v)
        pltpu.sync_copy(dst_hbm.at[pl.ds(wid * cpt, cpt)], idx_all)
        plsc.subcore_barrier()

        def chunk(j, carry):
            pltpu.sync_copy(ones_v, acc.at[idx_all.at[j]], add=True)
            return carry

        lax.fori_loop(0, cpt, chunk, 0)
        plsc.subcore_barrier()

        @pl.when(jnp.logical_and(sid == 0, cid == 0))
        def _():
            pltpu.sync_copy(acc, stg)
            pltpu.sync_copy(stg, deg0)

        @pl.when(jnp.logical_and(sid == 0, cid == 1))
        def _():
            pltpu.sync_copy(acc, stg)
            pltpu.sync_copy(stg, deg1)

    return pl.kernel(
        body,
        out_type=[jax.ShapeDtypeStruct((n,), jnp.float32),
                  jax.ShapeDtypeStruct((n,), jnp.float32)],
        mesh=_mesh,
        scratch_types=[
            pltpu.VMEM((cpt, _CH), jnp.int32),
            pltpu.VMEM((_CH,), jnp.float32),
            pltpu.VMEM((n,), jnp.float32),
            pltpu.VMEM_SHARED((n,), jnp.float32),
            pltpu.SemaphoreType.DMA,
        ],
    )


_WT = 10
_WC = 40


def _agg_kernel(n, e, d):
    cpt = e // _CH // _NW
    wr = n // _WT
    nwc = wr // _WC

    def body(src_hbm, dst_hbm, y_hbm, z2_hbm, out0, out1,
             idx_s, idx_d, buf, zbuf, acc, sem):
        cid = lax.axis_index("c")
        sid = lax.axis_index("s")
        wid = cid * _NS + sid

        @pl.when(sid < _WT)
        def _():
            pltpu.sync_copy(z2_hbm, zbuf)
            for k in range(nwc):
                pltpu.sync_copy(zbuf, acc.at[pl.ds(sid * wr + k * _WC, _WC)])

        pltpu.sync_copy(src_hbm.at[pl.ds(wid * cpt, cpt)], idx_s)
        pltpu.sync_copy(dst_hbm.at[pl.ds(wid * cpt, cpt)], idx_d)
        plsc.subcore_barrier()

        def chunk(j, carry):
            pltpu.async_copy(y_hbm.at[idx_s.at[j]], buf, sem).wait()
            pltpu.sync_copy(buf, acc.at[idx_d.at[j]], add=True)
            return carry

        lax.fori_loop(0, cpt, chunk, 0)
        plsc.subcore_barrier()

        @pl.when(sid < _WT)
        def _():
            for k in range(nwc):
                r = sid * wr + k * _WC
                pltpu.sync_copy(acc.at[pl.ds(r, _WC)], zbuf)

                @pl.when(cid == 0)
                def _():
                    pltpu.sync_copy(zbuf, out0.at[pl.ds(r, _WC)])

                @pl.when(cid == 1)
                def _():
                    pltpu.sync_copy(zbuf, out1.at[pl.ds(r, _WC)])

    return pl.kernel(
        body,
        out_type=[jax.ShapeDtypeStruct((n, d), jnp.float32),
                  jax.ShapeDtypeStruct((n, d), jnp.float32)],
        mesh=_mesh,
        scratch_types=[
            pltpu.VMEM((cpt, _CH), jnp.int32),
            pltpu.VMEM((cpt, _CH), jnp.int32),
            pltpu.VMEM((_CH, d), jnp.float32),
            pltpu.VMEM((_WC, d), jnp.float32),
            pltpu.VMEM_SHARED((n, d), jnp.float32),
            pltpu.SemaphoreType.DMA,
        ],
    )


def _y_body(x_ref, w_ref, d0_ref, d1_ref, y_ref, dinv_ref):
    deg = d0_ref[...] + d1_ref[...] + 1.0
    dv = lax.rsqrt(deg)
    xw = jnp.dot(x_ref[...], w_ref[...], preferred_element_type=jnp.float32)
    y_ref[...] = xw * dv
    dinv_ref[...] = dv


def _head_body(a0_ref, a1_ref, y_ref, dv_ref, bc_ref, wr_ref, br_ref, o_ref):
    s = a0_ref[...] + a1_ref[...] + y_ref[...]
    h = jnp.maximum(s * dv_ref[...] + bc_ref[...], 0.0)
    o_ref[...] = jnp.sum(h * wr_ref[...], axis=1, keepdims=True) + br_ref[...]


def kernel(x, edge_index, W_conv, b_conv, W_reg, b_reg):
    n, d = x.shape
    e = edge_index.shape[1]
    rows = e // _CH
    blk = n // 5

    src2 = edge_index[0].reshape(rows, _CH)
    dst2 = edge_index[1].reshape(rows, _CH)
    ones_ch = jnp.ones((_CH,), jnp.float32)
    z1 = jnp.zeros((n,), jnp.float32)
    z2 = jnp.zeros((_WC, d), jnp.float32)

    deg0, deg1 = _deg_kernel(n, e)(dst2, ones_ch, z1)

    y, dinv = pl.pallas_call(
        _y_body,
        grid=(n // blk,),
        in_specs=[
            pl.BlockSpec((blk, d), lambda i: (i, 0)),
            pl.BlockSpec((d, d), lambda i: (0, 0)),
            pl.BlockSpec((blk, 1), lambda i: (i, 0)),
            pl.BlockSpec((blk, 1), lambda i: (i, 0)),
        ],
        out_specs=[
            pl.BlockSpec((blk, d), lambda i: (i, 0)),
            pl.BlockSpec((blk, 1), lambda i: (i, 0)),
        ],
        out_shape=[
            jax.ShapeDtypeStruct((n, d), jnp.float32),
            jax.ShapeDtypeStruct((n, 1), jnp.float32),
        ],
    )(x, W_conv, deg0.reshape(n, 1), deg1.reshape(n, 1))

    acc0, acc1 = _agg_kernel(n, e, d)(src2, dst2, y, z2)

    out = pl.pallas_call(
        _head_body,
        grid=(n // blk,),
        in_specs=[
            pl.BlockSpec((blk, d), lambda i: (i, 0)),
            pl.BlockSpec((blk, d), lambda i: (i, 0)),
            pl.BlockSpec((blk, d), lambda i: (i, 0)),
            pl.BlockSpec((blk, 1), lambda i: (i, 0)),
            pl.BlockSpec((1, d), lambda i: (0, 0)),
            pl.BlockSpec((1, d), lambda i: (0, 0)),
            pl.BlockSpec((1, 1), lambda i: (0, 0)),
        ],
        out_specs=pl.BlockSpec((blk, 1), lambda i: (i, 0)),
        out_shape=jax.ShapeDtypeStruct((n, 1), jnp.float32),
    )(acc0, acc1, y, dinv, b_conv.reshape(1, d), W_reg.reshape(1, d),
      b_reg.reshape(1, 1))

    return out


# speedup vs baseline: 36.3819x; 1.2573x over previous
import jax
import jax.numpy as jnp
from jax import lax
from jax.experimental import pallas as pl
from jax.experimental.pallas import tpu as pltpu
from jax.experimental.pallas import tpu_sc as plsc

_NC = 2
_NS = 16
_NW = _NC * _NS
_CH = 125
_EPT = 10240
_CHA = 80
_NDUM = 8

_mesh = plsc.VectorSubcoreMesh(core_axis_name="c", subcore_axis_name="s")


def _deg_kernel(n, e):
    cpt = e // _CH // _NW

    def body(dst_hbm, ones_hbm, z1_hbm, deg0, deg1, idx_all, ones_v, stg, acc,
             sem):
        cid = lax.axis_index("c")
        sid = lax.axis_index("s")
        wid = cid * _NS + sid

        @pl.when(sid == 0)
        def _():
            pltpu.sync_copy(z1_hbm, stg)
            pltpu.sync_copy(stg, acc)

        pltpu.sync_copy(ones_hbm, ones_v)
        pltpu.sync_copy(dst_hbm.at[pl.ds(wid * cpt, cpt)], idx_all)
        plsc.subcore_barrier()

        def chunk(j, carry):
            pltpu.sync_copy(ones_v, acc.at[idx_all.at[j]], add=True)
            return carry

        lax.fori_loop(0, cpt, chunk, 0)
        plsc.subcore_barrier()

        @pl.when(jnp.logical_and(sid == 0, cid == 0))
        def _():
            pltpu.sync_copy(acc, stg)
            pltpu.sync_copy(stg, deg0)

        @pl.when(jnp.logical_and(sid == 0, cid == 1))
        def _():
            pltpu.sync_copy(acc, stg)
            pltpu.sync_copy(stg, deg1)

    return pl.kernel(
        body,
        out_type=[jax.ShapeDtypeStruct((n,), jnp.float32),
                  jax.ShapeDtypeStruct((n,), jnp.float32)],
        mesh=_mesh,
        scratch_types=[
            pltpu.VMEM((cpt, _CH), jnp.int32),
            pltpu.VMEM((_CH,), jnp.float32),
            pltpu.VMEM((n,), jnp.float32),
            pltpu.VMEM_SHARED((n,), jnp.float32),
            pltpu.SemaphoreType.DMA,
        ],
    )


_WT = 10
_WC = 40


def _agg_kernel(n, d):
    cpt = _EPT // _CHA
    wr = n // _WT
    nwc = wr // _WC

    def body(src_hbm, dst_hbm, y_hbm, z2_hbm, out0, out1,
             idx_s, idx_d, buf0, buf1, ds0, ds1, zbuf, acc, sem0, sem1):
        cid = lax.axis_index("c")
        sid = lax.axis_index("s")
        wid = cid * _NS + sid

        @pl.when(sid < _WT)
        def _():
            pltpu.sync_copy(z2_hbm, zbuf)
            for k in range(nwc):
                pltpu.sync_copy(zbuf, acc.at[pl.ds(sid * wr + k * _WC, _WC)])

        pltpu.sync_copy(src_hbm.at[pl.ds(wid * _EPT, _EPT)], idx_s)
        pltpu.sync_copy(dst_hbm.at[pl.ds(wid * _EPT, _EPT)], idx_d)
        plsc.subcore_barrier()

        def gather(j, buf, sem):
            pltpu.async_copy(y_hbm.at[idx_s.at[pl.ds(j * _CHA, _CHA)]],
                             buf, sem)

        def gwait(j, buf, sem):
            pltpu.make_async_copy(y_hbm.at[idx_s.at[pl.ds(j * _CHA, _CHA)]],
                                  buf, sem).wait()

        def stage(j, dsb):
            for k in range(_CHA // 16):
                dsb[pl.ds(k * 16, 16)] = idx_d[pl.ds(j * _CHA + k * 16, 16)]

        gather(0, buf0, sem0)
        stage(0, ds0)

        def chunk2(i, carry):
            j0 = 2 * i
            j1 = j0 + 1
            j2 = j0 + 2
            gather(j1, buf1, sem1)
            stage(j1, ds1)
            gwait(j0, buf0, sem0)
            pltpu.sync_copy(buf0, acc.at[ds0], add=True)

            @pl.when(j2 < cpt)
            def _():
                gather(j2, buf0, sem0)
                stage(j2, ds0)

            gwait(j1, buf1, sem1)
            pltpu.sync_copy(buf1, acc.at[ds1], add=True)
            return carry

        lax.fori_loop(0, cpt // 2, chunk2, 0)
        plsc.subcore_barrier()

        @pl.when(sid < _WT)
        def _():
            for k in range(nwc):
                r = sid * wr + k * _WC
                pltpu.sync_copy(acc.at[pl.ds(r, _WC)], zbuf)

                @pl.when(cid == 0)
                def _():
                    pltpu.sync_copy(zbuf, out0.at[pl.ds(r, _WC)])

                @pl.when(cid == 1)
                def _():
                    pltpu.sync_copy(zbuf, out1.at[pl.ds(r, _WC)])

    return pl.kernel(
        body,
        out_type=[jax.ShapeDtypeStruct((n, d), jnp.float32),
                  jax.ShapeDtypeStruct((n, d), jnp.float32)],
        mesh=_mesh,
        scratch_types=[
            pltpu.VMEM((_EPT,), jnp.int32),
            pltpu.VMEM((_EPT,), jnp.int32),
            pltpu.VMEM((_CHA, d), jnp.float32),
            pltpu.VMEM((_CHA, d), jnp.float32),
            pltpu.VMEM((_CHA,), jnp.int32),
            pltpu.VMEM((_CHA,), jnp.int32),
            pltpu.VMEM((_WC, d), jnp.float32),
            pltpu.VMEM_SHARED((n + _NDUM, d), jnp.float32),
            pltpu.SemaphoreType.DMA,
            pltpu.SemaphoreType.DMA,
        ],
    )


def _y_body(x_ref, w_ref, d0_ref, d1_ref, y_ref, dinv_ref):
    deg = d0_ref[...] + d1_ref[...] + 1.0
    dv = lax.rsqrt(deg)
    xw = jnp.dot(x_ref[...], w_ref[...], preferred_element_type=jnp.float32)
    y_ref[...] = xw * dv
    dinv_ref[...] = dv


def _head_body(a0_ref, a1_ref, y_ref, dv_ref, bc_ref, wr_ref, br_ref, o_ref):
    s = a0_ref[...] + a1_ref[...] + y_ref[...]
    h = jnp.maximum(s * dv_ref[...] + bc_ref[...], 0.0)
    o_ref[...] = jnp.sum(h * wr_ref[...], axis=1, keepdims=True) + br_ref[...]


def kernel(x, edge_index, W_conv, b_conv, W_reg, b_reg):
    n, d = x.shape
    e = edge_index.shape[1]
    blk = n // 5

    dst2 = edge_index[1].reshape(e // _CH, _CH)
    ones_ch = jnp.ones((_CH,), jnp.float32)
    z1 = jnp.zeros((n,), jnp.float32)
    z2 = jnp.zeros((_WC, d), jnp.float32)

    npad = _EPT * _NW - e
    ar = jnp.arange(npad, dtype=jnp.int32)
    src_p = jnp.concatenate([edge_index[0], (ar * 131) % n])
    dst_p = jnp.concatenate([edge_index[1], n + (ar % _NDUM)])

    deg0, deg1 = _deg_kernel(n, e)(dst2, ones_ch, z1)

    y, dinv = pl.pallas_call(
        _y_body,
        grid=(n // blk,),
        in_specs=[
            pl.BlockSpec((blk, d), lambda i: (i, 0)),
            pl.BlockSpec((d, d), lambda i: (0, 0)),
            pl.BlockSpec((blk, 1), lambda i: (i, 0)),
            pl.BlockSpec((blk, 1), lambda i: (i, 0)),
        ],
        out_specs=[
            pl.BlockSpec((blk, d), lambda i: (i, 0)),
            pl.BlockSpec((blk, 1), lambda i: (i, 0)),
        ],
        out_shape=[
            jax.ShapeDtypeStruct((n, d), jnp.float32),
            jax.ShapeDtypeStruct((n, 1), jnp.float32),
        ],
    )(x, W_conv, deg0.reshape(n, 1), deg1.reshape(n, 1))

    acc0, acc1 = _agg_kernel(n, d)(src_p, dst_p, y, z2)

    out = pl.pallas_call(
        _head_body,
        grid=(n // blk,),
        in_specs=[
            pl.BlockSpec((blk, d), lambda i: (i, 0)),
            pl.BlockSpec((blk, d), lambda i: (i, 0)),
            pl.BlockSpec((blk, d), lambda i: (i, 0)),
            pl.BlockSpec((blk, 1), lambda i: (i, 0)),
            pl.BlockSpec((1, d), lambda i: (0, 0)),
            pl.BlockSpec((1, d), lambda i: (0, 0)),
            pl.BlockSpec((1, 1), lambda i: (0, 0)),
        ],
        out_specs=pl.BlockSpec((blk, 1), lambda i: (i, 0)),
        out_shape=jax.ShapeDtypeStruct((n, 1), jnp.float32),
    )(acc0, acc1, y, dinv, b_conv.reshape(1, d), W_reg.reshape(1, d),
      b_reg.reshape(1, 1))

    return out


# speedup vs baseline: 41.7589x; 1.1478x over previous
import jax
import jax.numpy as jnp
from jax import lax
from jax.experimental import pallas as pl
from jax.experimental.pallas import tpu as pltpu
from jax.experimental.pallas import tpu_sc as plsc

_NC = 2
_NS = 16
_NW = _NC * _NS
_CH = 125
_NPH = 2

_mesh = plsc.VectorSubcoreMesh(core_axis_name="c", subcore_axis_name="s")


def _deg_kernel(n, e):
    cpt = e // _CH // _NW

    def body(ei_hbm, ones_hbm, z1_hbm, deg0, deg1, idx_all, ones_v, stg, acc,
             sem):
        cid = lax.axis_index("c")
        sid = lax.axis_index("s")
        wid = cid * _NS + sid

        @pl.when(sid == 0)
        def _():
            pltpu.sync_copy(z1_hbm, stg)
            pltpu.sync_copy(stg, acc)

        pltpu.sync_copy(ones_hbm, ones_v)
        pltpu.sync_copy(ei_hbm.at[1, pl.ds(wid * cpt, cpt)], idx_all)
        plsc.subcore_barrier()

        def chunk(j, carry):
            pltpu.sync_copy(ones_v, acc.at[idx_all.at[j]], add=True)
            return carry

        lax.fori_loop(0, cpt, chunk, 0)
        plsc.subcore_barrier()

        @pl.when(jnp.logical_and(sid == 0, cid == 0))
        def _():
            pltpu.sync_copy(acc, stg)
            pltpu.sync_copy(stg, deg0)

        @pl.when(jnp.logical_and(sid == 0, cid == 1))
        def _():
            pltpu.sync_copy(acc, stg)
            pltpu.sync_copy(stg, deg1)

    return pl.kernel(
        body,
        out_type=[jax.ShapeDtypeStruct((n,), jnp.float32),
                  jax.ShapeDtypeStruct((n,), jnp.float32)],
        mesh=_mesh,
        scratch_types=[
            pltpu.VMEM((cpt, _CH), jnp.int32),
            pltpu.VMEM((_CH,), jnp.float32),
            pltpu.VMEM((n,), jnp.float32),
            pltpu.VMEM_SHARED((n,), jnp.float32),
            pltpu.SemaphoreType.DMA,
        ],
    )


_WT = 10
_WC = 40


def _agg_kernel(n, e, d):
    cpt = e // _CH // _NW
    cps = cpt // _NPH
    wr = n // _WT
    nwc = wr // _WC

    def body(ei_hbm, y_hbm, z2_hbm, out0, out1,
             idx_s, idx_d, buf0, buf1, zbuf, acc, sem0, sem1):
        cid = lax.axis_index("c")
        sid = lax.axis_index("s")
        wid = cid * _NS + sid

        @pl.when(sid < _WT)
        def _():
            pltpu.sync_copy(z2_hbm, zbuf)
            for k in range(nwc):
                pltpu.sync_copy(zbuf, acc.at[pl.ds(sid * wr + k * _WC, _WC)])

        plsc.subcore_barrier()

        def gather(j, buf, sem):
            pltpu.async_copy(y_hbm.at[idx_s.at[j]], buf, sem)

        def gwait(j, buf, sem):
            pltpu.make_async_copy(y_hbm.at[idx_s.at[j]], buf, sem).wait()

        for p in range(_NPH):
            row0 = wid * cpt + p * cps
            pltpu.sync_copy(ei_hbm.at[0, pl.ds(row0, cps)], idx_s)
            pltpu.sync_copy(ei_hbm.at[1, pl.ds(row0, cps)], idx_d)
            gather(0, buf0, sem0)

            def chunk2(i, carry):
                j0 = 2 * i
                j1 = j0 + 1
                j2 = j0 + 2
                gather(j1, buf1, sem1)
                gwait(j0, buf0, sem0)
                pltpu.sync_copy(buf0, acc.at[idx_d.at[j0]], add=True)

                @pl.when(j2 < cps)
                def _():
                    gather(j2, buf0, sem0)

                gwait(j1, buf1, sem1)
                pltpu.sync_copy(buf1, acc.at[idx_d.at[j1]], add=True)
                return carry

            lax.fori_loop(0, cps // 2, chunk2, 0)

        plsc.subcore_barrier()

        @pl.when(sid < _WT)
        def _():
            for k in range(nwc):
                r = sid * wr + k * _WC
                pltpu.sync_copy(acc.at[pl.ds(r, _WC)], zbuf)

                @pl.when(cid == 0)
                def _():
                    pltpu.sync_copy(zbuf, out0.at[pl.ds(r, _WC)])

                @pl.when(cid == 1)
                def _():
                    pltpu.sync_copy(zbuf, out1.at[pl.ds(r, _WC)])

    return pl.kernel(
        body,
        out_type=[jax.ShapeDtypeStruct((n, d), jnp.float32),
                  jax.ShapeDtypeStruct((n, d), jnp.float32)],
        mesh=_mesh,
        scratch_types=[
            pltpu.VMEM((cpt // _NPH, _CH), jnp.int32),
            pltpu.VMEM((cpt // _NPH, _CH), jnp.int32),
            pltpu.VMEM((_CH, d), jnp.float32),
            pltpu.VMEM((_CH, d), jnp.float32),
            pltpu.VMEM((_WC, d), jnp.float32),
            pltpu.VMEM_SHARED((n, d), jnp.float32),
            pltpu.SemaphoreType.DMA,
            pltpu.SemaphoreType.DMA,
        ],
    )


def _dv_body(d0_ref, d1_ref, dv_ref):
    deg = d0_ref[...] + d1_ref[...] + 1.0
    dv_ref[...] = lax.rsqrt(deg)[:, None]


def _y_body(x_ref, w_ref, dv_ref, y_ref):
    xw = jnp.dot(x_ref[...], w_ref[...], preferred_element_type=jnp.float32)
    y_ref[...] = xw * dv_ref[...]


def _head_body(a0_ref, a1_ref, y_ref, dv_ref, bc_ref, wr_ref, br_ref, o_ref):
    s = a0_ref[...] + a1_ref[...] + y_ref[...]
    h = jnp.maximum(s * dv_ref[...] + bc_ref[...], 0.0)
    o_ref[...] = jnp.sum(h * wr_ref[...], axis=1, keepdims=True) + br_ref[...]


def kernel(x, edge_index, W_conv, b_conv, W_reg, b_reg):
    n, d = x.shape
    e = edge_index.shape[1]
    blk = n // 5

    ei3 = edge_index.reshape(2, e // _CH, _CH)
    ones_ch = jnp.ones((_CH,), jnp.float32)
    z1 = jnp.zeros((n,), jnp.float32)
    z2 = jnp.zeros((_WC, d), jnp.float32)

    deg0, deg1 = _deg_kernel(n, e)(ei3, ones_ch, z1)

    dinv = pl.pallas_call(
        _dv_body,
        out_shape=jax.ShapeDtypeStruct((n, 1), jnp.float32),
    )(deg0, deg1)

    y = pl.pallas_call(
        _y_body,
        grid=(n // blk,),
        in_specs=[
            pl.BlockSpec((blk, d), lambda i: (i, 0)),
            pl.BlockSpec((d, d), lambda i: (0, 0)),
            pl.BlockSpec((blk, 1), lambda i: (i, 0)),
        ],
        out_specs=pl.BlockSpec((blk, d), lambda i: (i, 0)),
        out_shape=jax.ShapeDtypeStruct((n, d), jnp.float32),
    )(x, W_conv, dinv)

    acc0, acc1 = _agg_kernel(n, e, d)(ei3, y, z2)

    out = pl.pallas_call(
        _head_body,
        grid=(n // blk,),
        in_specs=[
            pl.BlockSpec((blk, d), lambda i: (i, 0)),
            pl.BlockSpec((blk, d), lambda i: (i, 0)),
            pl.BlockSpec((blk, d), lambda i: (i, 0)),
            pl.BlockSpec((blk, 1), lambda i: (i, 0)),
            pl.BlockSpec((1, d), lambda i: (0, 0)),
            pl.BlockSpec((1, d), lambda i: (0, 0)),
            pl.BlockSpec((1, 1), lambda i: (0, 0)),
        ],
        out_specs=pl.BlockSpec((blk, 1), lambda i: (i, 0)),
        out_shape=jax.ShapeDtypeStruct((n, 1), jnp.float32),
    )(acc0, acc1, y, dinv, b_conv.reshape(1, d), W_reg.reshape(1, d),
      b_reg.reshape(1, 1))

    return out
